# baseline (speedup 1.0000x reference)
import functools

import jax
import jax.numpy as jnp
from jax.experimental import pallas as pl

Q = 1024
D = 64
K = 100000
KB = 2048
KPAD = 100352
NBLK = KPAD // KB
CAND = 50


def _sim_block(q_ref, k_ref, out_ref):
    q = q_ref[...]
    k = k_ref[...]
    qn = q / (jnp.sqrt(jnp.sum(q * q, axis=-1, keepdims=True)) + 1e-12)
    kn = k / (jnp.sqrt(jnp.sum(k * k, axis=-1, keepdims=True)) + 1e-12)
    out_ref[...] = jax.lax.dot_general(
        qn, kn, (((1,), (1,)), ((), ())),
        preferred_element_type=jnp.float32)


def kernel(queries, keys):
    keys_p = jnp.pad(keys, ((0, KPAD - K), (0, 0)))
    sim = pl.pallas_call(
        _sim_block,
        grid=(NBLK,),
        in_specs=[
            pl.BlockSpec((Q, D), lambda i: (0, 0)),
            pl.BlockSpec((KB, D), lambda i: (i, 0)),
        ],
        out_specs=pl.BlockSpec((Q, KB), lambda i: (0, i)),
        out_shape=jax.ShapeDtypeStruct((Q, KPAD), jnp.float32),
    )(queries, keys_p)
    vals, idx = jax.lax.top_k(sim[:, :K], CAND)
    return vals, idx


# speedup vs baseline: 4.3073x; 4.3073x over previous
import jax
import jax.numpy as jnp
from jax.experimental import pallas as pl

Q = 1024
D = 64
K = 100000
KB = 2048
KPAD = 102400
NBLK = KPAD // KB
R = 32
G = KPAD // R
GB = KB // R
GREAL = K // R
GSEL = 64
CW = GSEL * R
CAND = 50
BISECT_ITERS = 34
PAD_VAL = -2.0
FILL_VAL = -3.0


def _simgm_block(q_ref, k_ref, sim_ref, gm_ref):
    i = pl.program_id(0)
    q = q_ref[...]
    k = k_ref[...]
    qn = q / (jnp.sqrt(jnp.sum(q * q, axis=-1, keepdims=True)) + 1e-12)
    kn = k / (jnp.sqrt(jnp.sum(k * k, axis=-1, keepdims=True)) + 1e-12)
    sim = jax.lax.dot_general(
        qn, kn, (((1,), (1,)), ((), ())),
        preferred_element_type=jnp.float32)
    sim_ref[...] = sim
    bgm = jnp.max(sim.reshape(Q, GB, R), axis=-1)
    gid = jax.lax.broadcasted_iota(jnp.int32, (Q, GB), 1) + i * GB
    gm_ref[0] = jnp.where(gid < GREAL, bgm, PAD_VAL)


def _bisect(gm_ref, tau_ref):
    gm = gm_ref[...]

    def body(_, lohi):
        lo, hi = lohi
        mid = 0.5 * (lo + hi)
        ge = (gm >= mid[None, :, :]).astype(jnp.float32)
        cnt = jnp.sum(jnp.sum(ge, axis=-1), axis=0)[:, None]
        take = cnt >= float(CAND)
        return jnp.where(take, mid, lo), jnp.where(take, hi, mid)

    lo0 = jnp.full((Q, 1), -1.01, jnp.float32)
    hi0 = jnp.full((Q, 1), 1.01, jnp.float32)
    lo, _ = jax.lax.fori_loop(0, BISECT_ITERS, body, (lo0, hi0))
    tau_ref[...] = lo


def _extract_topk(vals_ref, idx_ref, ovals_ref, oidx_ref):
    v = vals_ref[...]
    ci = idx_ref[...]
    ocol = jax.lax.broadcasted_iota(jnp.int32, (Q, CAND), 1)
    big = jnp.int32(2**30)

    def body(t, carry):
        v, ov, oi = carry
        m = jnp.max(v, axis=1, keepdims=True)
        g = jnp.min(jnp.where(v == m, ci, big), axis=1, keepdims=True)
        v = jnp.where(ci == g, FILL_VAL, v)
        ov = jnp.where(ocol == t, m, ov)
        oi = jnp.where(ocol == t, g, oi)
        return v, ov, oi

    ov0 = jnp.zeros((Q, CAND), jnp.float32)
    oi0 = jnp.zeros((Q, CAND), jnp.int32)
    _, ov, oi = jax.lax.fori_loop(0, CAND, body, (v, ov0, oi0))
    ovals_ref[...] = ov
    oidx_ref[...] = oi


_INTERPRET = False


def _stage_a(queries, keys_p):
    return pl.pallas_call(
        _simgm_block,
        grid=(NBLK,),
        in_specs=[
            pl.BlockSpec((Q, D), lambda i: (0, 0)),
            pl.BlockSpec((KB, D), lambda i: (i, 0)),
        ],
        out_specs=[
            pl.BlockSpec((Q, KB), lambda i: (0, i)),
            pl.BlockSpec((1, Q, GB), lambda i: (i, 0, 0)),
        ],
        out_shape=[
            jax.ShapeDtypeStruct((Q, KPAD), jnp.float32),
            jax.ShapeDtypeStruct((NBLK, Q, GB), jnp.float32),
        ],
        interpret=_INTERPRET,
    )(queries, keys_p)


def _stage_a2(gm):
    return pl.pallas_call(
        _bisect,
        in_specs=[pl.BlockSpec((NBLK, Q, GB), lambda: (0, 0, 0))],
        out_specs=pl.BlockSpec((Q, 1), lambda: (0, 0)),
        out_shape=jax.ShapeDtypeStruct((Q, 1), jnp.float32),
        interpret=_INTERPRET,
    )(gm)


def _stage_d(cvals, cidx):
    return pl.pallas_call(
        _extract_topk,
        in_specs=[
            pl.BlockSpec((Q, CW), lambda: (0, 0)),
            pl.BlockSpec((Q, CW), lambda: (0, 0)),
        ],
        out_specs=[
            pl.BlockSpec((Q, CAND), lambda: (0, 0)),
            pl.BlockSpec((Q, CAND), lambda: (0, 0)),
        ],
        out_shape=[
            jax.ShapeDtypeStruct((Q, CAND), jnp.float32),
            jax.ShapeDtypeStruct((Q, CAND), jnp.int32),
        ],
        interpret=_INTERPRET,
    )(cvals, cidx)


def _stage_b_xla(sim, gmq, tau):
    sel = gmq >= tau
    gid = jnp.where(sel, jnp.arange(G, dtype=jnp.int32)[None, :], G - 1)
    order = jnp.argsort(jnp.where(sel, 0, 1), axis=1, stable=True)
    gsel = jnp.take_along_axis(gid, order[:, :GSEL], axis=1)
    sim3 = sim.reshape(Q, G, R)
    cvals = jnp.take_along_axis(sim3, gsel[:, :, None], axis=1)
    cidx = gsel[:, :, None] * R + jnp.arange(R, dtype=jnp.int32)[None, None, :]
    return cvals.reshape(Q, CW), cidx.reshape(Q, CW)


def kernel(queries, keys):
    keys_p = jnp.pad(keys, ((0, KPAD - K), (0, 0)))
    sim, gm = _stage_a(queries, keys_p)
    tau = _stage_a2(gm)
    gmq = jnp.swapaxes(gm, 0, 1).reshape(Q, G)
    cvals, cidx = _stage_b_xla(sim, gmq, tau)
    vals, idx = _stage_d(cvals, cidx)
    return vals, idx
